# baseline (speedup 1.0000x reference)
import functools
import math

import jax
import jax.numpy as jnp
from jax.experimental import pallas as pl

B, L, D = 16, 1, 4096
H, KVH, HD = 32, 8, 128
N_REP = H // KVH
KV = 2048


def _rotary(y, cs, ss):
    lane = jax.lax.broadcasted_iota(jnp.int32, y.shape, 2)
    partner = jnp.where(lane % 2 == 0,
                        jnp.roll(y, -1, axis=2),
                        jnp.roll(y, 1, axis=2))
    return y * cs + partner * ss


PW = 512
NPH = PW // HD


def _qkv_proj_kernel(x_ref, wq_ref, wk_ref, wv_ref, cs_ref, ss_ref,
                     q_ref, k_ref, v_ref):
    j = pl.program_id(0)
    x = x_ref[...]
    cs = cs_ref[...].reshape(1, 1, HD)
    ss = ss_ref[...].reshape(1, 1, HD)
    q = jnp.dot(x, wq_ref[...], preferred_element_type=jnp.float32)
    q_ref[...] = _rotary(q.reshape(B, NPH, HD), cs, ss).reshape(B, PW)

    @pl.when(j < (KVH * HD) // PW)
    def _():
        k = jnp.dot(x, wk_ref[...], preferred_element_type=jnp.float32)
        k_ref[...] = _rotary(k.reshape(B, NPH, HD), cs, ss).reshape(B, PW)
        v_ref[...] = jnp.dot(x, wv_ref[...], preferred_element_type=jnp.float32)


def _attn_one(q, kc, vc, kn, vn):
    scale = 1.0 / math.sqrt(HD)
    s = jax.lax.dot_general(q, kc, (((1,), (1,)), ((), ())),
                            preferred_element_type=jnp.float32) * scale
    sn = jax.lax.dot_general(q, kn, (((1,), (1,)), ((), ())),
                             preferred_element_type=jnp.float32) * scale
    m = jnp.maximum(jnp.max(s, axis=-1, keepdims=True), sn)
    p = jnp.exp(s - m)
    pn = jnp.exp(sn - m)
    denom = jnp.sum(p, axis=-1, keepdims=True) + pn
    p = p / denom
    pn = pn / denom
    o = jax.lax.dot_general(p, vc, (((1,), (0,)), ((), ())),
                            preferred_element_type=jnp.float32)
    return o + pn * vn


def _attn_kernel(q_ref, kc_ref, vc_ref, kn_ref, vn_ref, o_ref):
    for h in range(4):
        o_ref[0, h] = _attn_one(q_ref[0, h], kc_ref[0, h], vc_ref[0, h],
                                kn_ref[0, h], vn_ref[0, h])


def _out_proj_kernel(a_ref, wo_ref, o_ref):
    o_ref[...] = jnp.dot(a_ref[...], wo_ref[...],
                         preferred_element_type=jnp.float32)


@functools.partial(jax.jit, static_argnames=())
def kernel(x, start_pos, freqs_complex, k_cache, v_cache, wq, wk, wv, wo):
    del start_pos
    x2 = x.reshape(B, D)

    cos = freqs_complex[0, :, 0]
    sin = freqs_complex[0, :, 1]
    lane = jnp.arange(HD)
    cs = cos[lane // 2][None, :].astype(jnp.float32)
    ss = jnp.where(lane % 2 == 0, -sin[lane // 2], sin[lane // 2])[None, :]
    ss = ss.astype(jnp.float32)

    q2, k2, v2 = pl.pallas_call(
        _qkv_proj_kernel,
        grid=(H * HD // PW,),
        in_specs=[
            pl.BlockSpec((B, D), lambda j: (0, 0)),
            pl.BlockSpec((D, PW), lambda j: (0, j)),
            pl.BlockSpec((D, PW), lambda j: (0, jnp.minimum(j, KVH * HD // PW - 1))),
            pl.BlockSpec((D, PW), lambda j: (0, jnp.minimum(j, KVH * HD // PW - 1))),
            pl.BlockSpec((1, HD), lambda j: (0, 0)),
            pl.BlockSpec((1, HD), lambda j: (0, 0)),
        ],
        out_specs=[
            pl.BlockSpec((B, PW), lambda j: (0, j)),
            pl.BlockSpec((B, PW), lambda j: (0, jnp.minimum(j, KVH * HD // PW - 1))),
            pl.BlockSpec((B, PW), lambda j: (0, jnp.minimum(j, KVH * HD // PW - 1))),
        ],
        out_shape=[
            jax.ShapeDtypeStruct((B, H * HD), jnp.float32),
            jax.ShapeDtypeStruct((B, KVH * HD), jnp.float32),
            jax.ShapeDtypeStruct((B, KVH * HD), jnp.float32),
        ],
    )(x2, wq, wk, wv, cs, ss)

    qg = q2.reshape(B, KVH, N_REP, HD)
    kn = k2.reshape(B, KVH, 1, HD)
    vn = v2.reshape(B, KVH, 1, HD)

    attn = pl.pallas_call(
        _attn_kernel,
        grid=(B, KVH // 4),
        in_specs=[
            pl.BlockSpec((1, 4, N_REP, HD), lambda b, j: (b, j, 0, 0)),
            pl.BlockSpec((1, 4, KV, HD), lambda b, j: (b, j, 0, 0)),
            pl.BlockSpec((1, 4, KV, HD), lambda b, j: (b, j, 0, 0)),
            pl.BlockSpec((1, 4, 1, HD), lambda b, j: (b, j, 0, 0)),
            pl.BlockSpec((1, 4, 1, HD), lambda b, j: (b, j, 0, 0)),
        ],
        out_specs=pl.BlockSpec((1, 4, N_REP, HD), lambda b, j: (b, j, 0, 0)),
        out_shape=jax.ShapeDtypeStruct((B, KVH, N_REP, HD), jnp.float32),
    )(qg, k_cache, v_cache, kn, vn)

    a2 = attn.reshape(B, H * HD)
    out = pl.pallas_call(
        _out_proj_kernel,
        grid=(D // 1024,),
        in_specs=[
            pl.BlockSpec((B, H * HD), lambda j: (0, 0)),
            pl.BlockSpec((H * HD, 1024), lambda j: (0, j)),
        ],
        out_specs=pl.BlockSpec((B, 1024), lambda j: (0, j)),
        out_shape=jax.ShapeDtypeStruct((B, D), jnp.float32),
    )(a2, wo)

    return out.reshape(B, L, D)


# speedup vs baseline: 1.0134x; 1.0134x over previous
import functools
import math

import jax
import jax.numpy as jnp
from jax.experimental import pallas as pl

B, L, D = 16, 1, 4096
H, KVH, HD = 32, 8, 128
N_REP = H // KVH
KV = 2048


def _rotary(y, cs, ss):
    lane = jax.lax.broadcasted_iota(jnp.int32, y.shape, 2)
    partner = jnp.where(lane % 2 == 0,
                        jnp.roll(y, -1, axis=2),
                        jnp.roll(y, 1, axis=2))
    return y * cs + partner * ss


PW = 512
NPH = PW // HD


def _qkv_proj_kernel(x_ref, wq_ref, wk_ref, wv_ref, cs_ref, ss_ref,
                     q_ref, k_ref, v_ref):
    j = pl.program_id(0)
    x = x_ref[...]
    cs = cs_ref[...].reshape(1, 1, HD)
    ss = ss_ref[...].reshape(1, 1, HD)
    q = jnp.dot(x, wq_ref[...], preferred_element_type=jnp.float32)
    q_ref[...] = _rotary(q.reshape(B, NPH, HD), cs, ss).reshape(B, PW)

    @pl.when(j < (KVH * HD) // PW)
    def _():
        k = jnp.dot(x, wk_ref[...], preferred_element_type=jnp.float32)
        k_ref[...] = _rotary(k.reshape(B, NPH, HD), cs, ss).reshape(B, PW)
        v_ref[...] = jnp.dot(x, wv_ref[...], preferred_element_type=jnp.float32)


def _attn_one(qnew, kc, vc):
    q = qnew[:N_REP]
    kn = qnew[N_REP:N_REP + 1]
    vn = qnew[N_REP + 1:N_REP + 2]
    scale = 1.0 / math.sqrt(HD)
    s = jax.lax.dot_general(q, kc, (((1,), (1,)), ((), ())),
                            preferred_element_type=jnp.float32) * scale
    sn = jax.lax.dot_general(q, kn, (((1,), (1,)), ((), ())),
                             preferred_element_type=jnp.float32) * scale
    m = jnp.maximum(jnp.max(s, axis=-1, keepdims=True), sn)
    p = jnp.exp(s - m)
    pn = jnp.exp(sn - m)
    denom = jnp.sum(p, axis=-1, keepdims=True) + pn
    p = p / denom
    pn = pn / denom
    o = jax.lax.dot_general(p, vc, (((1,), (0,)), ((), ())),
                            preferred_element_type=jnp.float32)
    return o + pn * vn


def _attn_kernel(q_ref, kc_ref, vc_ref, o_ref):
    for h in range(4):
        o_ref[0, h] = _attn_one(q_ref[0, h], kc_ref[0, h], vc_ref[0, h])


def _out_proj_kernel(a_ref, wo_ref, o_ref):
    o_ref[...] = jnp.dot(a_ref[...], wo_ref[...],
                         preferred_element_type=jnp.float32)


@functools.partial(jax.jit, static_argnames=())
def kernel(x, start_pos, freqs_complex, k_cache, v_cache, wq, wk, wv, wo):
    del start_pos
    x2 = x.reshape(B, D)

    cos = freqs_complex[0, :, 0]
    sin = freqs_complex[0, :, 1]
    lane = jnp.arange(HD)
    cs = cos[lane // 2][None, :].astype(jnp.float32)
    ss = jnp.where(lane % 2 == 0, -sin[lane // 2], sin[lane // 2])[None, :]
    ss = ss.astype(jnp.float32)

    q2, k2, v2 = pl.pallas_call(
        _qkv_proj_kernel,
        grid=(H * HD // PW,),
        in_specs=[
            pl.BlockSpec((B, D), lambda j: (0, 0)),
            pl.BlockSpec((D, PW), lambda j: (0, j)),
            pl.BlockSpec((D, PW), lambda j: (0, jnp.minimum(j, KVH * HD // PW - 1))),
            pl.BlockSpec((D, PW), lambda j: (0, jnp.minimum(j, KVH * HD // PW - 1))),
            pl.BlockSpec((1, HD), lambda j: (0, 0)),
            pl.BlockSpec((1, HD), lambda j: (0, 0)),
        ],
        out_specs=[
            pl.BlockSpec((B, PW), lambda j: (0, j)),
            pl.BlockSpec((B, PW), lambda j: (0, jnp.minimum(j, KVH * HD // PW - 1))),
            pl.BlockSpec((B, PW), lambda j: (0, jnp.minimum(j, KVH * HD // PW - 1))),
        ],
        out_shape=[
            jax.ShapeDtypeStruct((B, H * HD), jnp.float32),
            jax.ShapeDtypeStruct((B, KVH * HD), jnp.float32),
            jax.ShapeDtypeStruct((B, KVH * HD), jnp.float32),
        ],
    )(x2, wq, wk, wv, cs, ss)

    qnew = jnp.concatenate([q2.reshape(B, KVH, N_REP, HD),
                            k2.reshape(B, KVH, 1, HD),
                            v2.reshape(B, KVH, 1, HD)], axis=2)

    attn = pl.pallas_call(
        _attn_kernel,
        grid=(B, KVH // 4),
        in_specs=[
            pl.BlockSpec((1, 4, N_REP + 2, HD), lambda b, j: (b, j, 0, 0)),
            pl.BlockSpec((1, 4, KV, HD), lambda b, j: (b, j, 0, 0)),
            pl.BlockSpec((1, 4, KV, HD), lambda b, j: (b, j, 0, 0)),
        ],
        out_specs=pl.BlockSpec((1, 4, N_REP, HD), lambda b, j: (b, j, 0, 0)),
        out_shape=jax.ShapeDtypeStruct((B, KVH, N_REP, HD), jnp.float32),
    )(qnew, k_cache, v_cache)

    a2 = attn.reshape(B, H * HD)
    out = pl.pallas_call(
        _out_proj_kernel,
        grid=(D // PW,),
        in_specs=[
            pl.BlockSpec((B, H * HD), lambda j: (0, 0)),
            pl.BlockSpec((H * HD, PW), lambda j: (0, j)),
        ],
        out_specs=pl.BlockSpec((B, PW), lambda j: (0, j)),
        out_shape=jax.ShapeDtypeStruct((B, D), jnp.float32),
    )(a2, wo)

    return out.reshape(B, L, D)


# speedup vs baseline: 1.0737x; 1.0594x over previous
import functools
import math

import jax
import jax.numpy as jnp
from jax.experimental import pallas as pl

B, L, D = 16, 1, 4096
H, KVH, HD = 32, 8, 128
N_REP = H // KVH
KV = 2048


def _rotary(y, cs, ss):
    lane = jax.lax.broadcasted_iota(jnp.int32, y.shape, 2)
    partner = jnp.where(lane % 2 == 0,
                        jnp.roll(y, -1, axis=2),
                        jnp.roll(y, 1, axis=2))
    return y * cs + partner * ss


PW = 512
NPH = PW // HD


def _qkv_proj_kernel(x_ref, wq_ref, wk_ref, wv_ref, cs_ref, ss_ref,
                     q_ref, k_ref, v_ref):
    j = pl.program_id(0)
    x = x_ref[...]
    cs = cs_ref[...].reshape(1, 1, HD)
    ss = ss_ref[...].reshape(1, 1, HD)
    q = jnp.dot(x, wq_ref[...], preferred_element_type=jnp.float32)
    q_ref[...] = _rotary(q.reshape(B, NPH, HD), cs, ss).reshape(B, PW)

    @pl.when(j < (KVH * HD) // PW)
    def _():
        k = jnp.dot(x, wk_ref[...], preferred_element_type=jnp.float32)
        k_ref[...] = _rotary(k.reshape(B, NPH, HD), cs, ss).reshape(B, PW)
        v_ref[...] = jnp.dot(x, wv_ref[...], preferred_element_type=jnp.float32)


def _attn_one(qnew, kc, vc):
    q = qnew[:N_REP]
    kn = qnew[N_REP:N_REP + 1]
    vn = qnew[N_REP + 1:N_REP + 2]
    scale = 1.0 / math.sqrt(HD)
    s = jax.lax.dot_general(q, kc, (((1,), (1,)), ((), ())),
                            preferred_element_type=jnp.float32) * scale
    sn = jax.lax.dot_general(q, kn, (((1,), (1,)), ((), ())),
                             preferred_element_type=jnp.float32) * scale
    m = jnp.maximum(jnp.max(s, axis=-1, keepdims=True), sn)
    p = jnp.exp(s - m)
    pn = jnp.exp(sn - m)
    denom = jnp.sum(p, axis=-1, keepdims=True) + pn
    p = p / denom
    pn = pn / denom
    o = jax.lax.dot_general(p, vc, (((1,), (0,)), ((), ())),
                            preferred_element_type=jnp.float32)
    return o + pn * vn


def _attn_kernel(q_ref, kc_ref, vc_ref, o_ref):
    for h in range(KVH):
        o_ref[0, h] = _attn_one(q_ref[0, h], kc_ref[0, h], vc_ref[0, h])


def _out_proj_kernel(a_ref, wo_ref, o_ref):
    o_ref[...] = jnp.dot(a_ref[...], wo_ref[...],
                         preferred_element_type=jnp.float32)


@functools.partial(jax.jit, static_argnames=())
def kernel(x, start_pos, freqs_complex, k_cache, v_cache, wq, wk, wv, wo):
    del start_pos
    x2 = x.reshape(B, D)

    cos = freqs_complex[0, :, 0]
    sin = freqs_complex[0, :, 1]
    lane = jnp.arange(HD)
    cs = cos[lane // 2][None, :].astype(jnp.float32)
    ss = jnp.where(lane % 2 == 0, -sin[lane // 2], sin[lane // 2])[None, :]
    ss = ss.astype(jnp.float32)

    q2, k2, v2 = pl.pallas_call(
        _qkv_proj_kernel,
        grid=(H * HD // PW,),
        in_specs=[
            pl.BlockSpec((B, D), lambda j: (0, 0)),
            pl.BlockSpec((D, PW), lambda j: (0, j)),
            pl.BlockSpec((D, PW), lambda j: (0, jnp.minimum(j, KVH * HD // PW - 1))),
            pl.BlockSpec((D, PW), lambda j: (0, jnp.minimum(j, KVH * HD // PW - 1))),
            pl.BlockSpec((1, HD), lambda j: (0, 0)),
            pl.BlockSpec((1, HD), lambda j: (0, 0)),
        ],
        out_specs=[
            pl.BlockSpec((B, PW), lambda j: (0, j)),
            pl.BlockSpec((B, PW), lambda j: (0, jnp.minimum(j, KVH * HD // PW - 1))),
            pl.BlockSpec((B, PW), lambda j: (0, jnp.minimum(j, KVH * HD // PW - 1))),
        ],
        out_shape=[
            jax.ShapeDtypeStruct((B, H * HD), jnp.float32),
            jax.ShapeDtypeStruct((B, KVH * HD), jnp.float32),
            jax.ShapeDtypeStruct((B, KVH * HD), jnp.float32),
        ],
    )(x2, wq, wk, wv, cs, ss)

    qnew = jnp.concatenate([q2.reshape(B, KVH, N_REP, HD),
                            k2.reshape(B, KVH, 1, HD),
                            v2.reshape(B, KVH, 1, HD)], axis=2)

    attn = pl.pallas_call(
        _attn_kernel,
        grid=(B,),
        in_specs=[
            pl.BlockSpec((1, KVH, N_REP + 2, HD), lambda b: (b, 0, 0, 0)),
            pl.BlockSpec((1, KVH, KV, HD), lambda b: (b, 0, 0, 0)),
            pl.BlockSpec((1, KVH, KV, HD), lambda b: (b, 0, 0, 0)),
        ],
        out_specs=pl.BlockSpec((1, KVH, N_REP, HD), lambda b: (b, 0, 0, 0)),
        out_shape=jax.ShapeDtypeStruct((B, KVH, N_REP, HD), jnp.float32),
    )(qnew, k_cache, v_cache)

    a2 = attn.reshape(B, H * HD)
    out = pl.pallas_call(
        _out_proj_kernel,
        grid=(D // PW,),
        in_specs=[
            pl.BlockSpec((B, H * HD), lambda j: (0, 0)),
            pl.BlockSpec((H * HD, PW), lambda j: (0, j)),
        ],
        out_specs=pl.BlockSpec((B, PW), lambda j: (0, j)),
        out_shape=jax.ShapeDtypeStruct((B, D), jnp.float32),
    )(a2, wo)

    return out.reshape(B, L, D)
